# baseline (speedup 1.0000x reference)
import functools
import math

import jax
import jax.numpy as jnp
from jax import lax
from jax.experimental import pallas as pl
from jax.experimental.pallas import tpu as pltpu
from jax.experimental.pallas import tpu_sc as plsc

VOCAB = 1000000
EMB = 64
SCALE = math.sqrt(EMB)

NC = 2
NS = 16
NW = NC * NS

CHUNK = 128
NBUF = 4
LANES = 16


def _body(tok_hbm, table_hbm, out_hbm, idx_v, rows_v, gsem):
  c = lax.axis_index("c")
  s = lax.axis_index("s")
  wid = s * NC + c
  nch = tok_hbm.shape[1]
  per_w = nch * CHUNK
  base = wid * per_w

  pltpu.sync_copy(tok_hbm.at[wid], idx_v)

  for b in range(NBUF):
    pltpu.async_copy(table_hbm.at[idx_v.at[b]], rows_v.at[b], gsem.at[b])

  def group(g, carry):
    for b in range(NBUF):
      j = g * NBUF + b
      pltpu.make_async_copy(
          table_hbm.at[idx_v.at[j]], rows_v.at[b], gsem.at[b]).wait()

      def scale_row(r, carry2):
        for e in range(EMB // LANES):
          sl = pl.ds(e * LANES, LANES)
          rows_v[b, r, sl] = rows_v[b, r, sl] * SCALE
        return carry2

      lax.fori_loop(0, CHUNK, scale_row, 0, unroll=2)

      pltpu.sync_copy(rows_v.at[b], out_hbm.at[pl.ds(base + j * CHUNK, CHUNK)])

      nj = j + NBUF

      @pl.when(nj < nch)
      def _():
        pltpu.async_copy(table_hbm.at[idx_v.at[nj]], rows_v.at[b], gsem.at[b])

    return carry

  lax.fori_loop(0, nch // NBUF, group, 0)


@jax.jit
def kernel(tokens, table):
  n = tokens.shape[0] * tokens.shape[1]
  assert n % (NW * CHUNK) == 0
  nch = n // (NW * CHUNK)
  idx = jnp.reshape(tokens.astype(jnp.int32), (NW, nch, CHUNK))

  mesh = plsc.VectorSubcoreMesh(
      core_axis_name="c", subcore_axis_name="s", num_cores=NC, num_subcores=NS)
  out = pl.kernel(
      _body,
      out_type=jax.ShapeDtypeStruct((n, EMB), jnp.float32),
      mesh=mesh,
      scratch_types=[
          pltpu.VMEM((nch, CHUNK), jnp.int32),
          pltpu.VMEM((NBUF, CHUNK, EMB), jnp.float32),
          pltpu.SemaphoreType.DMA((NBUF,)),
      ],
      compiler_params=pltpu.CompilerParams(use_tc_tiling_on_sc=False),
  )(idx, table)
  return jnp.reshape(out, (*tokens.shape, EMB))
